# baseline (speedup 1.0000x reference)
import functools

import jax
import jax.numpy as jnp
from jax import lax
from jax.experimental import pallas as pl
from jax.experimental.pallas import tpu as pltpu
from jax.experimental.pallas import tpu_sc as plsc

_D = 64
_WN = 128
_IB = 4
_HF = 2


def _body(x_ref, sp_ref, bc_ref, bt_ref, out_ref):
    pieces = {}
    for img in range(_IB):
        for c in range(3):
            xcb = x_ref[img, c].astype(jnp.bfloat16)
            xac = jax.lax.dot_general(
                sp_ref[...], xcb, (((1,), (0,)), ((), ())),
                preferred_element_type=jnp.float32).astype(jnp.bfloat16)
            for pi in range(4):
                for hf in range(_HF):
                    pieces[(img, c, pi, hf)] = jax.lax.slice(
                        xac, (pi * 64, hf * 128),
                        (pi * 64 + 64, hf * 128 + 128))

    for hf in range(_HF):
        rows = []
        for img in range(_IB):
            rows.append(jnp.concatenate(
                [pieces[(img, c, pi, hf)] for c in range(3)
                 for pi in range(4)], axis=1))
        lhs = jnp.concatenate(rows, axis=0)
        out4 = jax.lax.dot_general(
            lhs, bc_ref[...], (((1,), (0,)), ((), ())),
            preferred_element_type=jnp.float32)
        out4 = out4 + bt_ref[:, pl.ds(hf * 2048, 2048)]
        out_ref[:, :, pl.ds(hf * 2048, 2048)] = out4.reshape(_IB, 64, 2048)

def _idx_body(h_hbm, w_hbm, out_hbm, hv, wv, rowv):
    wid = lax.axis_index("s") * 2 + lax.axis_index("c")
    pltpu.sync_copy(h_hbm, hv)
    pltpu.sync_copy(w_hbm, wv)
    chunk = (wid * 2) >> 4
    hc = hv[pl.ds(chunk * 16, 16)]
    wc = wv[pl.ds(chunk * 16, 16)]
    offv = (hc >> 2) * _WN + (wc >> 2)
    for k in range(2):
        n = wid * 2 + k
        offs = lax.gather(
            offv, jnp.full((16, 1), n & 15, jnp.int32),
            lax.GatherDimensionNumbers(offset_dims=(),
                                       collapsed_slice_dims=(0,),
                                       start_index_map=(0,)),
            (1,), mode=lax.GatherScatterMode.PROMISE_IN_BOUNDS)

        def body(r, _):
            p = lax.iota(jnp.int32, 16) + r * 16
            rowv[pl.ds(r * 16, 16)] = (p >> 6) * _WN + (p & 63) + offs
            return _

        lax.fori_loop(0, 256, body, None)
        pltpu.sync_copy(rowv, out_hbm.at[n])


def kernel(x, h_offset, w_offset, W, b):
    N, H, Wd, C = x.shape
    h = H // 4
    w = Wd // 4
    xp = x.transpose(0, 3, 1, 2)

    rr = jnp.arange(256, dtype=jnp.int32)[:, None]
    cc = jnp.arange(256, dtype=jnp.int32)[None, :]
    sperm = (cc == (rr & 63) * 4 + (rr >> 6)).astype(jnp.bfloat16)

    w4 = W.reshape(4, 4, 3, _D)
    eye32 = jnp.eye(32, dtype=jnp.float32)
    bcat = jnp.einsum('pqco,jk->cpjqko', w4, eye32)
    bcat = bcat.reshape(1536, 2048).astype(jnp.bfloat16)

    bt = jnp.tile(b, w).reshape(1, w * _D)

    emb2 = pl.pallas_call(
        _body,
        grid=(N // _IB,),
        in_specs=[
            pl.BlockSpec((_IB, 3, H, Wd), lambda g: (g, 0, 0, 0)),
            pl.BlockSpec((256, 256), lambda g: (0, 0)),
            pl.BlockSpec((1536, 2048), lambda g: (0, 0)),
            pl.BlockSpec((1, w * _D), lambda g: (0, 0)),
        ],
        out_specs=pl.BlockSpec((_IB, h, w * _D), lambda g: (g, 0, 0)),
        out_shape=jax.ShapeDtypeStruct((N, h, w * _D), jnp.float32),
    )(xp, sperm, bcat, bt)

    idx_fn = functools.partial(
        pl.kernel,
        mesh=plsc.VectorSubcoreMesh(core_axis_name="c", subcore_axis_name="s"),
        out_type=jax.ShapeDtypeStruct((N, h * w), jnp.int32),
        scratch_types=[
            pltpu.VMEM((N,), jnp.int32),
            pltpu.VMEM((N,), jnp.int32),
            pltpu.VMEM((h * w,), jnp.int32),
        ],
    )(_idx_body)
    idx = idx_fn(h_offset, w_offset)
    return emb2.reshape(N, h * w, _D), idx
